# Initial kernel scaffold; baseline (speedup 1.0000x reference)
import jax
import jax.numpy as jnp
from jax.experimental import pallas as pl


def kernel(x_expert, x_team, edge_label_index_team_experts):
    raise NotImplementedError("write your pallas kernel here")


# speedup vs baseline: 4.9339x; 4.9339x over previous
import functools

import jax
import jax.numpy as jnp
from jax import lax
from jax.experimental import pallas as pl
from jax.experimental.pallas import tpu as pltpu
from jax.experimental.pallas import tpu_sc as plsc

B = 320000
D = 128
NC = 2
NS = 16
NW = NC * NS
BPW = B // NW
C = 400
NCHUNK = BPW // C
L = 16


def _sc_kernel(team_hbm, expert_hbm, ti_hbm, ei_hbm, out_hbm,
               ti_v, ei_v, trows, erows, outc, tbuf, s0, s1):
    wid = lax.axis_index("s") * NC + lax.axis_index("c")
    base0 = wid * BPW

    def chunk_body(i, carry):
        base = base0 + i * C
        pltpu.sync_copy(ti_hbm.at[pl.ds(base, C)], ti_v)
        pltpu.sync_copy(ei_hbm.at[pl.ds(base, C)], ei_v)
        cp0 = pltpu.async_copy(team_hbm.at[ti_v], trows, s0)
        cp1 = pltpu.async_copy(expert_hbm.at[ei_v], erows, s1)
        cp0.wait()
        cp1.wait()

        col_sc = lax.broadcasted_iota(jnp.int32, (L,), 0) * L

        def group_body(g, carry2):
            e0 = g * L
            for el in range(L):
                e = e0 + el
                acc = trows[e, pl.ds(0, L)] * erows[e, pl.ds(0, L)]
                for j in range(1, D // L):
                    acc = acc + (trows[e, pl.ds(j * L, L)]
                                 * erows[e, pl.ds(j * L, L)])
                plsc.store_scatter(tbuf, [col_sc + el], acc)
            tot = tbuf[pl.ds(0, L)]
            for l in range(1, L):
                tot = tot + tbuf[pl.ds(l * L, L)]
            outc[pl.ds(e0, L)] = tot
            return carry2

        lax.fori_loop(0, C // L, group_body, 0)
        pltpu.sync_copy(outc, out_hbm.at[pl.ds(base, C)])
        return carry

    lax.fori_loop(0, NCHUNK, chunk_body, 0)


@jax.jit
def _run(x_expert, x_team, team_idx, expert_idx):
    mesh = plsc.VectorSubcoreMesh(core_axis_name="c", subcore_axis_name="s")
    k = functools.partial(
        pl.kernel,
        out_type=jax.ShapeDtypeStruct((B,), jnp.float32),
        mesh=mesh,
        compiler_params=pltpu.CompilerParams(needs_layout_passes=False),
        scratch_types=[
            pltpu.VMEM((C,), jnp.int32),
            pltpu.VMEM((C,), jnp.int32),
            pltpu.VMEM((C, D), jnp.float32),
            pltpu.VMEM((C, D), jnp.float32),
            pltpu.VMEM((C,), jnp.float32),
            pltpu.VMEM((L * L,), jnp.float32),
            pltpu.SemaphoreType.DMA,
            pltpu.SemaphoreType.DMA,
        ],
    )(_sc_kernel)
    return k(x_team, x_expert, team_idx, expert_idx)


def kernel(x_expert, x_team, edge_label_index_team_experts):
    idx = edge_label_index_team_experts.astype(jnp.int32)
    return _run(x_expert, x_team, idx[0], idx[1])
